# Initial kernel scaffold; baseline (speedup 1.0000x reference)
import jax
import jax.numpy as jnp
from jax.experimental import pallas as pl


def kernel(x, indices, weight, bias):
    raise NotImplementedError("write your pallas kernel here")


# speedup vs baseline: 2.6059x; 2.6059x over previous
import jax
import jax.numpy as jnp
from jax.experimental import pallas as pl
from jax.experimental.pallas import tpu as pltpu


def _mm_kernel(wsel_ref, osel_ref, x_ref, w_ref, b_ref, o_ref):
    del wsel_ref, osel_ref
    acc = jax.lax.dot_general(
        x_ref[0], w_ref[0],
        dimension_numbers=(((1,), (1,)), ((), ())),
        preferred_element_type=jnp.float32,
    )
    o_ref[...] = (acc + b_ref[0])[None]


def kernel(x, indices, weight, bias):
    G, S = indices.shape
    E, OUT_D, IN_D = weight.shape
    T = x.shape[-2]
    P = G * S

    idx = indices.reshape(P)
    order = jnp.argsort(idx)
    wsel = jnp.take(idx, order)

    xr = x.reshape(G, T, IN_D)
    br = bias.reshape(E, 1, OUT_D)

    grid_spec = pltpu.PrefetchScalarGridSpec(
        num_scalar_prefetch=2,
        grid=(P,),
        in_specs=[
            pl.BlockSpec((1, T, IN_D),
                         lambda p, wsel, osel: (osel[p] // S, 0, 0)),
            pl.BlockSpec((1, OUT_D, IN_D),
                         lambda p, wsel, osel: (wsel[p], 0, 0)),
            pl.BlockSpec((1, 1, OUT_D),
                         lambda p, wsel, osel: (wsel[p], 0, 0)),
        ],
        out_specs=pl.BlockSpec((1, T, OUT_D),
                               lambda p, wsel, osel: (osel[p], 0, 0)),
    )

    out = pl.pallas_call(
        _mm_kernel,
        grid_spec=grid_spec,
        out_shape=jax.ShapeDtypeStruct((P, T, OUT_D), jnp.float32),
    )(wsel, order, xr, weight, br)

    return out.reshape(1, G, S, T, OUT_D)
